# baseline (speedup 1.0000x reference)
import functools

import jax
import jax.numpy as jnp
from jax import lax
from jax.experimental import pallas as pl
from jax.experimental.pallas import tpu as pltpu
from jax.experimental.pallas import tpu_sc as plsc

BATCH = 16384
EMBED_DIM = 64
PAIR_DIM = 2 * EMBED_DIM
NUM_CORES = 2
NUM_SUBCORES = 16
NUM_WORKERS = NUM_CORES * NUM_SUBCORES
B_PER_W = BATCH // NUM_WORKERS

COL_BLOCK = 8192
N_BLOCKS = 32
QH = N_BLOCKS * COL_BLOCK
QP = 2 * QH
LAST_COL_BLOCK = (1000001 - 1) // COL_BLOCK


def _pack_quads_tc(tt):
    def pack_kernel(a_ref, b_ref, c_ref, d_ref, eye_ref, o_ref):
        eye = eye_ref[...]

        def tbits(x_ref):
            t = lax.dot_general(
                x_ref[...].astype(jnp.bfloat16),
                eye,
                (((0,), (0,)), ((), ())),
                preferred_element_type=jnp.float32,
            )
            return lax.bitcast_convert_type(t, jnp.uint32)

        w0 = lax.bitcast_convert_type(
            (tbits(a_ref) >> 16) | tbits(b_ref), jnp.float32
        )
        w1 = lax.bitcast_convert_type(
            (tbits(c_ref) >> 16) | tbits(d_ref), jnp.float32
        )
        o_ref[...] = jnp.concatenate([w0, w1], axis=1)

    def clamped(off):
        return lambda i: (0, jnp.minimum(i + off, LAST_COL_BLOCK))

    return pl.pallas_call(
        pack_kernel,
        grid=(N_BLOCKS,),
        in_specs=[
            pl.BlockSpec((EMBED_DIM, COL_BLOCK), lambda i: (0, i)),
            pl.BlockSpec((EMBED_DIM, COL_BLOCK), clamped(2 * N_BLOCKS)),
            pl.BlockSpec((EMBED_DIM, COL_BLOCK), clamped(N_BLOCKS)),
            pl.BlockSpec((EMBED_DIM, COL_BLOCK), clamped(3 * N_BLOCKS)),
            pl.BlockSpec((EMBED_DIM, EMBED_DIM), lambda i: (0, 0)),
        ],
        out_specs=pl.BlockSpec((COL_BLOCK, PAIR_DIM), lambda i: (i, 0)),
        out_shape=jax.ShapeDtypeStruct((QH, PAIR_DIM), jnp.float32),
        compiler_params=pltpu.CompilerParams(
            dimension_semantics=("parallel",),
        ),
    )(tt, tt, tt, tt, jnp.eye(EMBED_DIM, dtype=jnp.bfloat16))


def _gather_quads_sc(quads, idx):
    mesh = plsc.VectorSubcoreMesh(core_axis_name="c", subcore_axis_name="s")

    @functools.partial(
        pl.kernel,
        mesh=mesh,
        out_type=jax.ShapeDtypeStruct((BATCH, PAIR_DIM), jnp.float32),
        scratch_types=[
            pltpu.VMEM((B_PER_W,), jnp.int32),
            pltpu.VMEM((B_PER_W, PAIR_DIM), jnp.float32),
            pltpu.SemaphoreType.DMA,
        ],
    )
    def gather_kernel(quads_hbm, idx_hbm, out_hbm, idx_v, rows_v, sem):
        wid = lax.axis_index("s") * NUM_CORES + lax.axis_index("c")
        base = wid * B_PER_W
        pltpu.sync_copy(idx_hbm.at[pl.ds(base, B_PER_W)], idx_v)
        pltpu.async_copy(quads_hbm.at[idx_v], rows_v, sem).wait()
        pltpu.sync_copy(rows_v, out_hbm.at[pl.ds(base, B_PER_W)])

    return gather_kernel(quads, idx)


def _project_tc(emb4, hsel, psel, Wt, b):
    block_b = 2048

    def proj_kernel(x_ref, h_ref, p_ref, wt_ref, b_ref, o_ref):
        w = lax.bitcast_convert_type(x_ref[...], jnp.uint32)
        hh = h_ref[...] != 0
        sel32 = jnp.where(hh, w[:, EMBED_DIM:], w[:, :EMBED_DIM])
        pp = p_ref[...] != 0
        bits = jnp.where(pp, sel32 >> 16, sel32 & 0xFFFF).astype(jnp.uint16)
        eb = lax.bitcast_convert_type(bits, jnp.bfloat16)
        o_ref[...] = (
            jnp.dot(eb, wt_ref[...], preferred_element_type=jnp.float32)
            + b_ref[...]
        )

    return pl.pallas_call(
        proj_kernel,
        grid=(BATCH // block_b,),
        in_specs=[
            pl.BlockSpec((block_b, PAIR_DIM), lambda i: (i, 0)),
            pl.BlockSpec((block_b, 1), lambda i: (i, 0)),
            pl.BlockSpec((block_b, 1), lambda i: (i, 0)),
            pl.BlockSpec((EMBED_DIM, EMBED_DIM), lambda i: (0, 0)),
            pl.BlockSpec((1, EMBED_DIM), lambda i: (0, 0)),
        ],
        out_specs=pl.BlockSpec((block_b, EMBED_DIM), lambda i: (i, 0)),
        out_shape=jax.ShapeDtypeStruct((BATCH, EMBED_DIM), jnp.float32),
    )(emb4, hsel, psel, Wt, b)


@jax.jit
def kernel(user_ids, table, W, b):
    ids = user_ids.astype(jnp.int32)
    quads = _pack_quads_tc(table.T)
    emb4 = _gather_quads_sc(quads, ids % QH)
    hsel = ((ids // QH) & 1).reshape(BATCH, 1)
    psel = (ids // QP).reshape(BATCH, 1)
    return _project_tc(
        emb4, hsel, psel, W.T.astype(jnp.bfloat16), b.reshape(1, EMBED_DIM)
    )


# speedup vs baseline: 1.0384x; 1.0384x over previous
import functools

import jax
import jax.numpy as jnp
from jax import lax
from jax.experimental import pallas as pl
from jax.experimental.pallas import tpu as pltpu
from jax.experimental.pallas import tpu_sc as plsc

BATCH = 16384
EMBED_DIM = 64
PAIR_DIM = 2 * EMBED_DIM
NUM_CORES = 2
NUM_SUBCORES = 16
NUM_WORKERS = NUM_CORES * NUM_SUBCORES
B_PER_W = BATCH // NUM_WORKERS

COL_BLOCK = 16384
N_BLOCKS = 16
QH = N_BLOCKS * COL_BLOCK
QP = 2 * QH
LAST_COL_BLOCK = (1000001 - 1) // COL_BLOCK


def _pack_quads_tc(tt):
    def pack_kernel(a_ref, b_ref, c_ref, d_ref, eye_ref, o_ref):
        eye = eye_ref[...]

        def tbits(x_ref):
            t = lax.dot_general(
                x_ref[...].astype(jnp.bfloat16),
                eye,
                (((0,), (0,)), ((), ())),
                preferred_element_type=jnp.float32,
            )
            return lax.bitcast_convert_type(t, jnp.uint32)

        w0 = lax.bitcast_convert_type(
            (tbits(a_ref) >> 16) | tbits(b_ref), jnp.float32
        )
        w1 = lax.bitcast_convert_type(
            (tbits(c_ref) >> 16) | tbits(d_ref), jnp.float32
        )
        o_ref[:, :EMBED_DIM] = w0
        o_ref[:, EMBED_DIM:] = w1

    def clamped(off):
        return lambda i: (0, jnp.minimum(i + off, LAST_COL_BLOCK))

    return pl.pallas_call(
        pack_kernel,
        grid=(N_BLOCKS,),
        in_specs=[
            pl.BlockSpec((EMBED_DIM, COL_BLOCK), lambda i: (0, i)),
            pl.BlockSpec((EMBED_DIM, COL_BLOCK), clamped(2 * N_BLOCKS)),
            pl.BlockSpec((EMBED_DIM, COL_BLOCK), clamped(N_BLOCKS)),
            pl.BlockSpec((EMBED_DIM, COL_BLOCK), clamped(3 * N_BLOCKS)),
            pl.BlockSpec((EMBED_DIM, EMBED_DIM), lambda i: (0, 0)),
        ],
        out_specs=pl.BlockSpec((COL_BLOCK, PAIR_DIM), lambda i: (i, 0)),
        out_shape=jax.ShapeDtypeStruct((QH, PAIR_DIM), jnp.float32),
        compiler_params=pltpu.CompilerParams(
            dimension_semantics=("parallel",),
        ),
    )(tt, tt, tt, tt, jnp.eye(EMBED_DIM, dtype=jnp.bfloat16))


def _gather_quads_sc(quads, idx):
    mesh = plsc.VectorSubcoreMesh(core_axis_name="c", subcore_axis_name="s")

    @functools.partial(
        pl.kernel,
        mesh=mesh,
        out_type=jax.ShapeDtypeStruct((BATCH, PAIR_DIM), jnp.float32),
        scratch_types=[
            pltpu.VMEM((B_PER_W,), jnp.int32),
            pltpu.VMEM((B_PER_W, PAIR_DIM), jnp.float32),
            pltpu.SemaphoreType.DMA,
        ],
    )
    def gather_kernel(quads_hbm, idx_hbm, out_hbm, idx_v, rows_v, sem):
        wid = lax.axis_index("s") * NUM_CORES + lax.axis_index("c")
        base = wid * B_PER_W
        pltpu.sync_copy(idx_hbm.at[pl.ds(base, B_PER_W)], idx_v)
        pltpu.async_copy(quads_hbm.at[idx_v], rows_v, sem).wait()
        pltpu.sync_copy(rows_v, out_hbm.at[pl.ds(base, B_PER_W)])

    return gather_kernel(quads, idx)


def _project_tc(emb4, hsel, psel, Wt, b):
    block_b = 2048

    def proj_kernel(x_ref, h_ref, p_ref, wt_ref, b_ref, o_ref):
        w = lax.bitcast_convert_type(x_ref[...], jnp.uint32)
        hh = h_ref[...] != 0
        sel32 = jnp.where(hh, w[:, EMBED_DIM:], w[:, :EMBED_DIM])
        pp = p_ref[...] != 0
        bits = jnp.where(pp, sel32 >> 16, sel32 & 0xFFFF).astype(jnp.uint16)
        eb = lax.bitcast_convert_type(bits, jnp.bfloat16)
        o_ref[...] = (
            jnp.dot(eb, wt_ref[...], preferred_element_type=jnp.float32)
            + b_ref[...]
        )

    return pl.pallas_call(
        proj_kernel,
        grid=(BATCH // block_b,),
        in_specs=[
            pl.BlockSpec((block_b, PAIR_DIM), lambda i: (i, 0)),
            pl.BlockSpec((block_b, 1), lambda i: (i, 0)),
            pl.BlockSpec((block_b, 1), lambda i: (i, 0)),
            pl.BlockSpec((EMBED_DIM, EMBED_DIM), lambda i: (0, 0)),
            pl.BlockSpec((1, EMBED_DIM), lambda i: (0, 0)),
        ],
        out_specs=pl.BlockSpec((block_b, EMBED_DIM), lambda i: (i, 0)),
        out_shape=jax.ShapeDtypeStruct((BATCH, EMBED_DIM), jnp.float32),
    )(emb4, hsel, psel, Wt, b)


@jax.jit
def kernel(user_ids, table, W, b):
    ids = user_ids.astype(jnp.int32)
    quads = _pack_quads_tc(table.T)
    emb4 = _gather_quads_sc(quads, ids % QH)
    hsel = ((ids // QH) & 1).reshape(BATCH, 1)
    psel = (ids // QP).reshape(BATCH, 1)
    return _project_tc(
        emb4, hsel, psel, W.T.astype(jnp.bfloat16), b.reshape(1, EMBED_DIM)
    )
